# Initial kernel scaffold; baseline (speedup 1.0000x reference)
import jax
import jax.numpy as jnp
from jax.experimental import pallas as pl


def kernel(center, normal, feature, W0, b0, gamma0, beta0, W1, b1, gamma1, beta1, W2, b2, gamma2, beta2):
    raise NotImplementedError("write your pallas kernel here")


# speedup vs baseline: 9.2185x; 9.2185x over previous
import functools

import jax
import jax.numpy as jnp
import numpy as np
from jax.experimental import pallas as pl
from jax.experimental.pallas import tpu as pltpu
from jax.experimental.pallas import tpu_sc as plsc

NPOINT = 1024
RADIUS = 0.3
NSAMPLE = 32
B, N, CF = 8, 4096, 32
TBL_W = 128


def _fps_body(x_ref, y_ref, z_ref, o_ref):
    x = x_ref[...]
    y = y_ref[...]
    z = z_ref[...]
    lane = jax.lax.broadcasted_iota(jnp.int32, (B, N), 1)
    slot = jax.lax.broadcasted_iota(jnp.int32, (B, NPOINT), 1)
    boff = jax.lax.broadcasted_iota(jnp.int32, (B, 1), 0) * N

    def step(i, carry):
        dist, far, idxs = carry
        idxs = jnp.where(slot == i, far + boff, idxs)
        sel = lane == far
        cx = jnp.sum(jnp.where(sel, x, 0.0), axis=1, keepdims=True)
        cy = jnp.sum(jnp.where(sel, y, 0.0), axis=1, keepdims=True)
        cz = jnp.sum(jnp.where(sel, z, 0.0), axis=1, keepdims=True)
        dx = x - cx
        dy = y - cy
        dz = z - cz
        d = dx * dx + dy * dy + dz * dz
        dist = jnp.minimum(dist, d)
        mx = jnp.max(dist, axis=1, keepdims=True)
        far = jnp.min(jnp.where(dist == mx, lane, N), axis=1, keepdims=True)
        return dist, far, idxs

    init = (jnp.full((B, N), 1e10, jnp.float32),
            jnp.zeros((B, 1), jnp.int32),
            jnp.zeros((B, NPOINT), jnp.int32))
    _, _, idxs = jax.lax.fori_loop(0, NPOINT, step, init)
    o_ref[...] = idxs


def _fps(center):
    return pl.pallas_call(
        _fps_body,
        out_shape=jax.ShapeDtypeStruct((B, NPOINT), jnp.int32),
    )(center[:, 0, :], center[:, 1, :], center[:, 2, :])


def _sc_gather(table, flat_idx, window=128):
    num = flat_idx.shape[0]
    assert num % window == 0
    idx2 = flat_idx.reshape(1, num)
    mesh = plsc.VectorSubcoreMesh(core_axis_name="core",
                                  subcore_axis_name="subcore")

    @pl.kernel(out_type=jax.ShapeDtypeStruct((num, TBL_W), table.dtype),
               mesh=mesh)
    def gather_kernel(x_hbm, i_hbm, o_hbm):
        def body(i_vmem, o_vmem):
            pltpu.sync_copy(x_hbm.at[i_vmem.at[0]], o_vmem)

        pltpu.emit_pipeline(
            body,
            grid=(num // window,),
            in_specs=[pl.BlockSpec((1, window), index_map=lambda i: (0, i))],
            out_specs=[pl.BlockSpec((window, TBL_W),
                                    index_map=lambda i: (i, 0))],
            core_axis_name=("core", "subcore"),
            dimension_semantics=(pltpu.PARALLEL,),
        )(i_hbm, o_hbm)

    return gather_kernel(table, idx2)


def _bq_body(q_ref, c_ref, o_ref, *, ts):
    b = pl.program_id(0)
    q = q_ref[0]
    c = c_ref[0]
    qn = jnp.sum(q * q, axis=1, keepdims=True)
    cn = jnp.sum(c * c, axis=0, keepdims=True)
    d = jax.lax.dot_general(q, c, (((1,), (0,)), ((), ())),
                            preferred_element_type=jnp.float32)
    sqr = qn + cn - 2.0 * d
    lane = jax.lax.broadcasted_iota(jnp.int32, (ts, N), 1)
    masked = jnp.where(sqr > RADIUS * RADIUS, N, lane)
    boff = b * N
    first = jnp.min(masked, axis=1, keepdims=True)
    o_ref[0, :, 0:1] = first + boff
    masked = jnp.where(masked == first, N, masked)
    for j in range(1, NSAMPLE):
        m = jnp.min(masked, axis=1, keepdims=True)
        o_ref[0, :, j:j + 1] = jnp.where(m == N, first, m) + boff
        masked = jnp.where(masked == m, N, masked)


def _ball_query(new_center, center, ts=256):
    grid = (B, NPOINT // ts)
    return pl.pallas_call(
        functools.partial(_bq_body, ts=ts),
        grid=grid,
        in_specs=[
            pl.BlockSpec((1, ts, 3), lambda b, t: (b, t, 0)),
            pl.BlockSpec((1, 3, N), lambda b, t: (b, 0, 0)),
        ],
        out_specs=pl.BlockSpec((1, ts, NSAMPLE), lambda b, t: (b, t, 0)),
        out_shape=jax.ShapeDtypeStruct((B, NPOINT, NSAMPLE), jnp.int32),
        compiler_params=pltpu.CompilerParams(
            dimension_semantics=("parallel", "parallel")),
    )(new_center, center)


def _build_x(g, nc, ts):
    gc = g[:, :, 0:3]
    gcn = gc - nc[:, None, :]
    rho = jnp.sqrt(jnp.sum(gcn * gcn, axis=-1, keepdims=True) + 1e-12)
    zc = jnp.clip(gcn[:, :, 2:3] / rho, -0.999999, 0.999999)
    theta = (jnp.arctan2(jnp.sqrt(jnp.maximum(1.0 - zc * zc, 0.0)), zc)
             * np.float32(1.0 / np.pi))
    phi = (jnp.arctan2(gcn[:, :, 1:2], gcn[:, :, 0:1])
           * np.float32(1.0 / (2.0 * np.pi)) + 0.5)
    x = jnp.concatenate([gcn, rho, theta, phi, g[:, :, 6:TBL_W]], axis=-1)
    return x.reshape(ts * NSAMPLE, TBL_W)


def _layer(h, wt, bb, scale, shift):
    y = jax.lax.dot_general(h, wt[...], (((1,), (0,)), ((), ())),
                            preferred_element_type=jnp.float32) + bb[...]
    if scale is None:
        return y
    return jnp.maximum(y * scale[...] + shift[...], 0.0)


def _stats_body(g_ref, nc_ref, *refs, ts, nlayers):
    o_ref, refs = refs[-1], refs[:-1]
    x = _build_x(g_ref[...], nc_ref[...], ts)
    h = x
    i = 0
    for l in range(nlayers - 1):
        h = _layer(h, refs[i], refs[i + 1], refs[i + 2], refs[i + 3])
        i += 4
    y = _layer(h, refs[i], refs[i + 1], None, None)
    s = jnp.sum(y, axis=0, keepdims=True)
    q = jnp.sum(y * y, axis=0, keepdims=True)
    o_ref[0] = jnp.concatenate([s, q], axis=0)


def _final_body(g_ref, nc_ref, *refs, ts):
    o_ref, refs = refs[-1], refs[:-1]
    x = _build_x(g_ref[...], nc_ref[...], ts)
    h = x
    for l in range(3):
        h = _layer(h, refs[4 * l], refs[4 * l + 1], refs[4 * l + 2],
                   refs[4 * l + 3])
    h = h.reshape(ts, NSAMPLE, h.shape[-1])
    o_ref[...] = jnp.max(h, axis=1)


def _mlp_pass(grouped, nc_flat, params, nlayers, final, ts=128):
    rows = B * NPOINT
    ntiles = rows // ts
    cout = params[-2].shape[1]
    in_specs = [
        pl.BlockSpec((ts, NSAMPLE, TBL_W), lambda t: (t, 0, 0)),
        pl.BlockSpec((ts, 3), lambda t: (t, 0)),
    ]
    for p in params:
        in_specs.append(
            pl.BlockSpec(p.shape, lambda t, nd=p.ndim: (0,) * nd))
    if final:
        body = functools.partial(_final_body, ts=ts)
        out_specs = pl.BlockSpec((ts, cout), lambda t: (t, 0))
        out_shape = jax.ShapeDtypeStruct((rows, cout), jnp.float32)
    else:
        body = functools.partial(_stats_body, ts=ts, nlayers=nlayers)
        out_specs = pl.BlockSpec((1, 2, cout), lambda t: (t, 0, 0))
        out_shape = jax.ShapeDtypeStruct((ntiles, 2, cout), jnp.float32)
    return pl.pallas_call(
        body,
        grid=(ntiles,),
        in_specs=in_specs,
        out_specs=out_specs,
        out_shape=out_shape,
        compiler_params=pltpu.CompilerParams(
            dimension_semantics=("parallel",)),
    )(grouped, nc_flat, *params)


def _bn_coeffs(stats, gamma, beta):
    tot = jnp.sum(stats, axis=0)
    cnt = np.float32(B * NPOINT * NSAMPLE)
    mean = tot[0] / cnt
    var = tot[1] / cnt - mean * mean
    scale = gamma / jnp.sqrt(var + 1e-5)
    shift = beta - mean * scale
    return scale[None, :], shift[None, :]


def kernel(center, normal, feature, W0, b0, gamma0, beta0,
           W1, b1, gamma1, beta1, W2, b2, gamma2, beta2):
    f32 = jnp.float32
    zeros3 = jnp.zeros((B, 3, N), f32)
    zeros87 = jnp.zeros((B, TBL_W - 41, N), f32)
    table = jnp.concatenate([center, zeros3, normal, feature, zeros87], axis=1)
    table = jnp.transpose(table, (0, 2, 1)).reshape(B * N, TBL_W)

    fps_idx = _fps(center)
    qrows = _sc_gather(table, fps_idx.reshape(-1))
    new_center = qrows[:, 0:3].reshape(B, NPOINT, 3)
    new_normal = qrows[:, 6:9].reshape(B, NPOINT, 3)

    gidx = _ball_query(new_center, center)
    grouped = _sc_gather(table, gidx.reshape(-1))
    grouped = grouped.reshape(B * NPOINT, NSAMPLE, TBL_W)
    nc_flat = new_center.reshape(B * NPOINT, 3)

    wt0 = jnp.zeros((TBL_W, 64), f32).at[:41].set(W0.T)
    wt1 = W1.T
    wt2 = W2.T
    b0r, b1r, b2r = b0[None, :], b1[None, :], b2[None, :]

    st0 = _mlp_pass(grouped, nc_flat, [wt0, b0r], 1, final=False)
    sc0, sh0 = _bn_coeffs(st0, gamma0, beta0)
    st1 = _mlp_pass(grouped, nc_flat, [wt0, b0r, sc0, sh0, wt1, b1r], 2,
                    final=False)
    sc1, sh1 = _bn_coeffs(st1, gamma1, beta1)
    st2 = _mlp_pass(grouped, nc_flat,
                    [wt0, b0r, sc0, sh0, wt1, b1r, sc1, sh1, wt2, b2r], 3,
                    final=False)
    sc2, sh2 = _bn_coeffs(st2, gamma2, beta2)
    out = _mlp_pass(grouped, nc_flat,
                    [wt0, b0r, sc0, sh0, wt1, b1r, sc1, sh1, wt2, b2r,
                     sc2, sh2], 3, final=True)

    new_center = jnp.transpose(new_center, (0, 2, 1))
    new_normal = jnp.transpose(new_normal, (0, 2, 1))
    x = jnp.transpose(out.reshape(B, NPOINT, 128), (0, 2, 1))
    return new_center, new_normal, x
